# initial kernel scaffold (unmeasured)
import jax
import jax.numpy as jnp
from jax import lax
from jax.experimental import pallas as pl
from jax.experimental.pallas import tpu as pltpu

N_DEV = 16


def _silu(y):
    return y * jax.nn.sigmoid(y)


def kernel(x, w_mat):
    m_per, k = x.shape
    k2, n_per = w_mat.shape
    assert k == k2
    M = N_DEV * m_per

    def body(x_hbm, w_ref, out_ref, comm_hbm,
             xin_sem, send_sems, recv_sems, stage_ref, stage_sems):
        my = lax.axis_index("i")
        left = (my - 1) % N_DEV
        right = (my + 1) % N_DEV

        barrier = pltpu.get_barrier_semaphore()
        for nbr in (left, right):
            pl.semaphore_signal(barrier, inc=1, device_id=(nbr,),
                                device_id_type=pl.DeviceIdType.MESH)
        pl.semaphore_wait(barrier, 2)

        cp_comm = pltpu.make_async_copy(x_hbm, comm_hbm.at[my], xin_sem)
        cp_comm.start()
        cp_stage = pltpu.make_async_copy(x_hbm, stage_ref.at[0], stage_sems.at[0])
        cp_stage.start()
        cp_comm.wait()
        cp_stage.wait()

        out_ref[pl.ds(my * m_per, m_per), :] = _silu(
            jnp.dot(stage_ref[0], w_ref[...], preferred_element_type=jnp.float32)
        )

        for h in range(N_DEV - 1):
            src_o = (my - h) % N_DEV
            dst_o = (my - h - 1) % N_DEV
            rdma = pltpu.make_async_remote_copy(
                src_ref=comm_hbm.at[src_o],
                dst_ref=comm_hbm.at[src_o],
                send_sem=send_sems.at[h],
                recv_sem=recv_sems.at[h],
                device_id=(right,),
                device_id_type=pl.DeviceIdType.MESH,
            )
            rdma.start()
            rdma.wait()

            cp = pltpu.make_async_copy(
                comm_hbm.at[dst_o], stage_ref.at[0], stage_sems.at[0]
            )
            cp.start()
            cp.wait()
            out_ref[pl.ds(dst_o * m_per, m_per), :] = _silu(
                jnp.dot(stage_ref[0], w_ref[...],
                        preferred_element_type=jnp.float32)
            )

    return pl.pallas_call(
        body,
        out_shape=jax.ShapeDtypeStruct((M, n_per), jnp.float32),
        in_specs=[
            pl.BlockSpec(memory_space=pltpu.MemorySpace.HBM),
            pl.BlockSpec(memory_space=pltpu.MemorySpace.VMEM),
        ],
        out_specs=pl.BlockSpec(memory_space=pltpu.MemorySpace.VMEM),
        scratch_shapes=[
            pltpu.HBM((N_DEV, m_per, k), jnp.float32),
            pltpu.SemaphoreType.DMA,
            pltpu.SemaphoreType.DMA((N_DEV - 1,)),
            pltpu.SemaphoreType.DMA((N_DEV - 1,)),
            pltpu.VMEM((2, m_per, k), jnp.float32),
            pltpu.SemaphoreType.DMA((2,)),
        ],
        compiler_params=pltpu.CompilerParams(collective_id=0),
    )(x, w_mat)


# baseline (device time: 3413902 ns/iter reference)
import jax
import jax.numpy as jnp
from jax import lax
from jax.experimental import pallas as pl
from jax.experimental.pallas import tpu as pltpu

N_DEV = 16


def _silu(y):
    return y * jax.nn.sigmoid(y)


def kernel(x, w_mat):
    m_per, k = x.shape
    k2, n_per = w_mat.shape
    assert k == k2
    M = N_DEV * m_per

    def body(x_hbm, w_ref, out_ref, comm_hbm,
             xin_sem, send_sems, recv_sems, stage_ref, stage_sems):
        my = lax.axis_index("i")
        left = (my - 1) % N_DEV
        right = (my + 1) % N_DEV

        barrier = pltpu.get_barrier_semaphore()
        for nbr in (left, right):
            pl.semaphore_signal(barrier, inc=1, device_id=(nbr,),
                                device_id_type=pl.DeviceIdType.MESH)
        pl.semaphore_wait(barrier, 2)

        cp_comm = pltpu.make_async_copy(x_hbm, comm_hbm.at[my], xin_sem)
        cp_comm.start()
        cp_stage = pltpu.make_async_copy(x_hbm, stage_ref.at[0], stage_sems.at[0])
        cp_stage.start()
        cp_comm.wait()
        cp_stage.wait()

        out_ref[pl.ds(my * m_per, m_per), :] = _silu(
            jnp.dot(stage_ref[0], w_ref[...], preferred_element_type=jnp.float32)
        )

        for h in range(N_DEV - 1):
            src_o = (my - h) % N_DEV
            dst_o = (my - h - 1) % N_DEV
            rdma = pltpu.make_async_remote_copy(
                src_ref=comm_hbm.at[src_o],
                dst_ref=comm_hbm.at[src_o],
                send_sem=send_sems.at[h],
                recv_sem=recv_sems.at[h],
                device_id=(right,),
                device_id_type=pl.DeviceIdType.MESH,
            )
            rdma.start()
            rdma.wait()

            cp = pltpu.make_async_copy(
                comm_hbm.at[dst_o], stage_ref.at[0], stage_sems.at[0]
            )
            cp.start()
            cp.wait()
            out_ref[pl.ds(dst_o * m_per, m_per), :] = _silu(
                jnp.dot(stage_ref[0], w_ref[...],
                        preferred_element_type=jnp.float32)
            )

    out, _comm = pl.pallas_call(
        body,
        out_shape=[
            jax.ShapeDtypeStruct((M, n_per), jnp.float32),
            jax.ShapeDtypeStruct((N_DEV, m_per, k), jnp.float32),
        ],
        in_specs=[
            pl.BlockSpec(memory_space=pltpu.MemorySpace.HBM),
            pl.BlockSpec(memory_space=pltpu.MemorySpace.VMEM),
        ],
        out_specs=[
            pl.BlockSpec(memory_space=pltpu.MemorySpace.VMEM),
            pl.BlockSpec(memory_space=pltpu.MemorySpace.HBM),
        ],
        scratch_shapes=[
            pltpu.SemaphoreType.DMA,
            pltpu.SemaphoreType.DMA((N_DEV - 1,)),
            pltpu.SemaphoreType.DMA((N_DEV - 1,)),
            pltpu.VMEM((2, m_per, k), jnp.float32),
            pltpu.SemaphoreType.DMA((2,)),
        ],
        compiler_params=pltpu.CompilerParams(collective_id=0),
    )(x, w_mat)
    return out


# device time: 1948382 ns/iter; 1.7522x vs baseline; 1.7522x over previous
import jax
import jax.numpy as jnp
from jax import lax
from jax.experimental import pallas as pl
from jax.experimental.pallas import tpu as pltpu

N_DEV = 16


def _silu(y):
    return y * jax.nn.sigmoid(y)


def kernel(x, w_mat):
    m_per, k = x.shape
    k2, n_per = w_mat.shape
    assert k == k2
    M = N_DEV * m_per
    half = m_per // 2

    def body(x_hbm, w_ref, out_ref, comm_hbm,
             xin_sem, send_r, recv_r, send_l, recv_l,
             stage_ref, stage_sems):
        my = lax.axis_index("i")
        left = (my - 1) % N_DEV
        right = (my + 1) % N_DEV

        TOP = pl.ds(0, half)
        BOT = pl.ds(half, half)

        def mk_r(h, origin):
            return pltpu.make_async_remote_copy(
                src_ref=comm_hbm.at[origin, TOP],
                dst_ref=comm_hbm.at[origin, TOP],
                send_sem=send_r.at[h],
                recv_sem=recv_r.at[h],
                device_id=(right,),
                device_id_type=pl.DeviceIdType.MESH,
            )

        def mk_l(h, origin):
            return pltpu.make_async_remote_copy(
                src_ref=comm_hbm.at[origin, BOT],
                dst_ref=comm_hbm.at[origin, BOT],
                send_sem=send_l.at[h],
                recv_sem=recv_l.at[h],
                device_id=(left,),
                device_id_type=pl.DeviceIdType.MESH,
            )

        def compute(origin, half_slice, stage_slot, out_row):
            cp = pltpu.make_async_copy(
                comm_hbm.at[origin, half_slice],
                stage_ref.at[stage_slot],
                stage_sems.at[stage_slot],
            )
            cp.start()
            cp.wait()
            out_ref[pl.ds(out_row, half), :] = _silu(
                jnp.dot(stage_ref[stage_slot], w_ref[...],
                        preferred_element_type=jnp.float32)
            )

        barrier = pltpu.get_barrier_semaphore()
        for nbr in (left, right):
            pl.semaphore_signal(barrier, inc=1, device_id=(nbr,),
                                device_id_type=pl.DeviceIdType.MESH)
        pl.semaphore_wait(barrier, 2)

        cp_comm = pltpu.make_async_copy(x_hbm, comm_hbm.at[my], xin_sem)
        cp_comm.start()
        cp_comm.wait()
        mk_r(0, my).start()
        mk_l(0, my).start()

        compute(my, TOP, 0, my * m_per)
        compute(my, BOT, 2, my * m_per + half)

        for h in range(N_DEV - 1):
            o_r = (my - 1 - h) % N_DEV
            o_l = (my + 1 + h) % N_DEV
            mk_r(h, o_r).wait_recv()
            mk_l(h, o_l).wait_recv()
            if h < N_DEV - 2:
                mk_r(h, (my - h) % N_DEV).wait_send()
                mk_l(h, (my + h) % N_DEV).wait_send()
                mk_r(h + 1, o_r).start()
                mk_l(h + 1, o_l).start()
            compute(o_r, TOP, h % 2, o_r * m_per)
            compute(o_l, BOT, 2 + h % 2, o_l * m_per + half)

        mk_r(N_DEV - 2, (my - (N_DEV - 2)) % N_DEV).wait_send()
        mk_l(N_DEV - 2, (my + (N_DEV - 2)) % N_DEV).wait_send()

    out, _comm = pl.pallas_call(
        body,
        out_shape=[
            jax.ShapeDtypeStruct((M, n_per), jnp.float32),
            jax.ShapeDtypeStruct((N_DEV, m_per, k), jnp.float32),
        ],
        in_specs=[
            pl.BlockSpec(memory_space=pltpu.MemorySpace.HBM),
            pl.BlockSpec(memory_space=pltpu.MemorySpace.VMEM),
        ],
        out_specs=[
            pl.BlockSpec(memory_space=pltpu.MemorySpace.VMEM),
            pl.BlockSpec(memory_space=pltpu.MemorySpace.HBM),
        ],
        scratch_shapes=[
            pltpu.SemaphoreType.DMA,
            pltpu.SemaphoreType.DMA((N_DEV - 1,)),
            pltpu.SemaphoreType.DMA((N_DEV - 1,)),
            pltpu.SemaphoreType.DMA((N_DEV - 1,)),
            pltpu.SemaphoreType.DMA((N_DEV - 1,)),
            pltpu.VMEM((4, half, k), jnp.float32),
            pltpu.SemaphoreType.DMA((4,)),
        ],
        compiler_params=pltpu.CompilerParams(
            collective_id=0,
            vmem_limit_bytes=100 * 1024 * 1024,
        ),
    )(x, w_mat)
    return out
